# Initial kernel scaffold; baseline (speedup 1.0000x reference)
import jax
import jax.numpy as jnp
from jax.experimental import pallas as pl


def kernel(videos, params):
    raise NotImplementedError("write your pallas kernel here")


# speedup vs baseline: 1.3323x; 1.3323x over previous
import functools

import jax
import jax.numpy as jnp
from jax.experimental import pallas as pl
from jax.experimental.pallas import tpu as pltpu

_IN_DIM = 3
_D = 512
_LATENT = 32
_NLAT = 8192
_PATCH = 16
_NBLK = 2
_NH = 8
_ADIM = 7
_B, _T, _H, _W = 4, 8, 224, 224
_PTD = _IN_DIM * _PATCH * _PATCH
_N = (_H // _PATCH) * (_W // _PATCH) + 1
_HD = _D // _NH
_SCALE = _HD ** -0.5


def _dot(a, b):
    return jax.lax.dot_general(
        a.astype(jnp.bfloat16), b.astype(jnp.bfloat16),
        (((a.ndim - 1,), (0,)), ((), ())),
        preferred_element_type=jnp.float32)


def _dot_nt(a, b):
    return jax.lax.dot_general(
        a.astype(jnp.bfloat16), b.astype(jnp.bfloat16),
        (((1,), (1,)), ((), ())),
        preferred_element_type=jnp.float32)


def _ln(x, s, b):
    mu = x.mean(-1, keepdims=True)
    var = ((x - mu) ** 2).mean(-1, keepdims=True)
    return (x - mu) / jnp.sqrt(var + 1e-6) * s + b


def _bf(x):
    return x.astype(jnp.bfloat16).astype(jnp.float32)


def _in_proj_body(p_ref, w_ref, b_ref, o_ref):
    p = p_ref[0, 0]
    o_ref[0, 0] = _dot(p, w_ref[...]) + b_ref[0]


def _spatial_body(x_ref, s_ref, bb_ref, wqkv_ref, wo_ref, o_ref):
    x = x_ref[0, 0]
    h = _ln(x, s_ref[0], bb_ref[0])
    qkv = _dot(h, wqkv_ref[...])
    outs = []
    for hh in range(_NH):
        q = qkv[:, hh * _HD:(hh + 1) * _HD]
        k = qkv[:, _D + hh * _HD:_D + (hh + 1) * _HD]
        v = qkv[:, 2 * _D + hh * _HD:2 * _D + (hh + 1) * _HD]
        s = _dot_nt(q, k) * _SCALE
        a = jax.nn.softmax(s, axis=-1)
        outs.append(_dot(a, v))
    o = jnp.concatenate(outs, axis=1)
    o_ref[0, 0] = x + _dot(o, wo_ref[...])


def _temporal_body(x_ref, s_ref, bb_ref, wqkv_ref, wo_ref, o_ref):
    x = x_ref[0]
    h = _ln(x, s_ref[0], bb_ref[0])
    qkv = _dot(h.reshape(_T * _N, _D), wqkv_ref[...]).reshape(_T, _N, 3 * _D)
    qr = _bf(qkv[:, :, :_D])
    kr = _bf(qkv[:, :, _D:2 * _D])
    vr = _bf(qkv[:, :, 2 * _D:])
    o_rows = []
    for t in range(_T):
        cols = []
        for u in range(t + 1):
            prod = qr[t] * kr[u]
            cols.append(prod.reshape(_N, _NH, _HD).sum(-1))
        s = jnp.stack(cols, axis=-1) * _SCALE
        a = _bf(jax.nn.softmax(s, axis=-1))
        o_t = jnp.zeros((_N, _D), jnp.float32)
        for u in range(t + 1):
            w = jnp.broadcast_to(a[:, :, u:u + 1], (_N, _NH, _HD))
            o_t = o_t + w.reshape(_N, _D) * vr[u]
        o_rows.append(o_t)
    o = jnp.stack(o_rows).reshape(_T * _N, _D)
    o_ref[0] = x + _dot(o, wo_ref[...]).reshape(_T, _N, _D)


def _mlp_body(x_ref, s_ref, bb_ref, w1_ref, b1_ref, w2_ref, b2_ref, o_ref):
    x = x_ref[0].reshape(_T * _N, _D)
    h = _ln(x, s_ref[0], bb_ref[0])
    a = jax.nn.gelu(_dot(h, w1_ref[...]) + b1_ref[0])
    o_ref[0] = (x + _dot(a, w2_ref[...]) + b2_ref[0]).reshape(_T, _N, _D)


def _spatial_q0_body(x_ref, s_ref, bb_ref, wq_ref, wkv_ref, wo_ref, o_ref):
    x = x_ref[0, 0]
    h = _ln(x, s_ref[0], bb_ref[0])
    kv = _dot(h, wkv_ref[...])
    q0 = _dot(h[0:1], wq_ref[...])
    outs = []
    for hh in range(_NH):
        q = q0[:, hh * _HD:(hh + 1) * _HD]
        k = kv[:, hh * _HD:(hh + 1) * _HD]
        v = kv[:, _D + hh * _HD:_D + (hh + 1) * _HD]
        s = _dot_nt(q, k) * _SCALE
        a = jax.nn.softmax(s, axis=-1)
        outs.append(_dot(a, v))
    o = jnp.concatenate(outs, axis=1)
    o_ref[0, 0] = x[0:1] + _dot(o, wo_ref[...])


def _tail_body(xa_ref, pen_ref, s2_ref, b2_ref, wqkv_ref, wo_ref,
               s3_ref, b3_ref, w1_ref, bm1_ref, w2_ref, bm2_ref,
               wout_ref, bout_ref, cb_ref, wact_ref, out_ref):
    x = xa_ref[...].reshape(_B * _T, _D)
    h = _ln(x, s2_ref[0], b2_ref[0])
    qkv = _dot(h, wqkv_ref[...])
    pen = pen_ref[...]
    o_rows = []
    for b in range(_B):
        qkv_b = qkv[b * _T:(b + 1) * _T]
        outs = []
        for hh in range(_NH):
            q = qkv_b[:, hh * _HD:(hh + 1) * _HD]
            k = qkv_b[:, _D + hh * _HD:_D + (hh + 1) * _HD]
            v = qkv_b[:, 2 * _D + hh * _HD:2 * _D + (hh + 1) * _HD]
            s = _dot_nt(q, k) * _SCALE + pen
            a = jax.nn.softmax(s, axis=-1)
            outs.append(_dot(a, v))
        o_rows.append(jnp.concatenate(outs, axis=1))
    o = jnp.concatenate(o_rows, axis=0)
    x2 = x + _dot(o, wo_ref[...])
    h3 = _ln(x2, s3_ref[0], b3_ref[0])
    m = jax.nn.gelu(_dot(h3, w1_ref[...]) + bm1_ref[0])
    x3 = x2 + _dot(m, w2_ref[...]) + bm2_ref[0]
    z = _dot(x3, wout_ref[...]) + bout_ref[0]
    z28 = jnp.concatenate(
        [z[b * _T + 1:(b + 1) * _T] for b in range(_B)], axis=0)
    cb = cb_ref[...]
    flat_sq = jnp.sum(z28 * z28, axis=-1, keepdims=True)
    cb_sq = jnp.transpose(jnp.sum(cb * cb, axis=-1, keepdims=True))
    d = (flat_sq - 2.0 * _dot_nt(z28, cb)) + cb_sq
    ii = jax.lax.broadcasted_iota(jnp.int32, (28, _NLAT), 1)
    dmin = jnp.min(d, axis=-1, keepdims=True)
    idx = jnp.min(jnp.where(d == dmin, ii, _NLAT), axis=-1)
    oh = (ii == idx[:, None]).astype(jnp.float32)
    zq = _dot(oh, cb)
    out_ref[...] = _dot(zq, wact_ref[...])


def _patches(videos):
    b, t, h, w, c = videos.shape
    hp, wp = h // _PATCH, w // _PATCH
    x = videos.reshape(b, t, hp, _PATCH, wp, _PATCH, c)
    x = x.transpose(0, 1, 2, 4, 3, 5, 6)
    return x.reshape(b, t, hp * wp, _PATCH * _PATCH * c)


def _full_spec(shape):
    nd = len(shape)
    return pl.BlockSpec(shape, lambda *_: (0,) * nd)


def kernel(videos, params):
    p = params
    bf = jnp.bfloat16
    patches = _patches(videos)
    act = jnp.broadcast_to(p['action_in'], (_B, _T, 1, _PTD))
    pin = jnp.concatenate([act, patches], axis=2)

    row_spec = pl.BlockSpec((1, 1, _N, _PTD), lambda b, t: (b, t, 0, 0))
    x_spec = pl.BlockSpec((1, 1, _N, _D), lambda b, t: (b, t, 0, 0))
    xt_spec = pl.BlockSpec((1, _T, _N, _D), lambda b: (b, 0, 0, 0))

    x = pl.pallas_call(
        _in_proj_body,
        grid=(_B, _T),
        in_specs=[row_spec,
                  _full_spec((_PTD, _D)),
                  _full_spec((1, _D))],
        out_specs=x_spec,
        out_shape=jax.ShapeDtypeStruct((_B, _T, _N, _D), jnp.float32),
    )(pin, p['W_in'].astype(bf), p['b_in'].reshape(1, _D))

    def spatial(x, i):
        return pl.pallas_call(
            _spatial_body,
            grid=(_B, _T),
            in_specs=[x_spec,
                      _full_spec((1, _D)), _full_spec((1, _D)),
                      _full_spec((_D, 3 * _D)), _full_spec((_D, _D))],
            out_specs=x_spec,
            out_shape=jax.ShapeDtypeStruct((_B, _T, _N, _D), jnp.float32),
        )(x, p['ln1_s'][i].reshape(1, _D), p['ln1_b'][i].reshape(1, _D),
          p['Ws_qkv'][i].astype(bf), p['Ws_o'][i].astype(bf))

    def temporal(x, i):
        return pl.pallas_call(
            _temporal_body,
            grid=(_B,),
            in_specs=[xt_spec,
                      _full_spec((1, _D)), _full_spec((1, _D)),
                      _full_spec((_D, 3 * _D)), _full_spec((_D, _D))],
            out_specs=xt_spec,
            out_shape=jax.ShapeDtypeStruct((_B, _T, _N, _D), jnp.float32),
        )(x, p['ln2_s'][i].reshape(1, _D), p['ln2_b'][i].reshape(1, _D),
          p['Wt_qkv'][i].astype(bf), p['Wt_o'][i].astype(bf))

    def mlp(x, i):
        return pl.pallas_call(
            _mlp_body,
            grid=(_B,),
            in_specs=[xt_spec,
                      _full_spec((1, _D)), _full_spec((1, _D)),
                      _full_spec((_D, 4 * _D)), _full_spec((1, 4 * _D)),
                      _full_spec((4 * _D, _D)), _full_spec((1, _D))],
            out_specs=xt_spec,
            out_shape=jax.ShapeDtypeStruct((_B, _T, _N, _D), jnp.float32),
        )(x, p['ln3_s'][i].reshape(1, _D), p['ln3_b'][i].reshape(1, _D),
          p['W_mlp1'][i].astype(bf), p['b_mlp1'][i].reshape(1, 4 * _D),
          p['W_mlp2'][i].astype(bf), p['b_mlp2'][i].reshape(1, _D))

    x = spatial(x, 0)
    x = temporal(x, 0)
    x = mlp(x, 0)

    xa = pl.pallas_call(
        _spatial_q0_body,
        grid=(_B, _T),
        in_specs=[x_spec,
                  _full_spec((1, _D)), _full_spec((1, _D)),
                  _full_spec((_D, _D)), _full_spec((_D, 2 * _D)),
                  _full_spec((_D, _D))],
        out_specs=pl.BlockSpec((1, 1, 1, _D), lambda b, t: (b, t, 0, 0)),
        out_shape=jax.ShapeDtypeStruct((_B, _T, 1, _D), jnp.float32),
    )(x, p['ln1_s'][1].reshape(1, _D), p['ln1_b'][1].reshape(1, _D),
      p['Ws_qkv'][1][:, :_D].astype(bf), p['Ws_qkv'][1][:, _D:].astype(bf),
      p['Ws_o'][1].astype(bf))

    wact_pad = jnp.zeros((_LATENT, 16), jnp.float32).at[:, :_ADIM].set(
        p['W_action'])
    pen = jnp.where(jnp.tril(jnp.ones((_T, _T), jnp.bool_)),
                    0.0, -1e9).astype(jnp.float32)
    out28 = pl.pallas_call(
        _tail_body,
        grid=(1,),
        in_specs=[_full_spec((_B, _T, 1, _D)),
                  _full_spec((_T, _T)),
                  _full_spec((1, _D)), _full_spec((1, _D)),
                  _full_spec((_D, 3 * _D)), _full_spec((_D, _D)),
                  _full_spec((1, _D)), _full_spec((1, _D)),
                  _full_spec((_D, 4 * _D)), _full_spec((1, 4 * _D)),
                  _full_spec((4 * _D, _D)), _full_spec((1, _D)),
                  _full_spec((_D, _LATENT)), _full_spec((1, _LATENT)),
                  _full_spec((_NLAT, _LATENT)), _full_spec((_LATENT, 16))],
        out_specs=_full_spec((28, 16)),
        out_shape=jax.ShapeDtypeStruct((28, 16), jnp.float32),
    )(xa, pen,
      p['ln2_s'][1].reshape(1, _D), p['ln2_b'][1].reshape(1, _D),
      p['Wt_qkv'][1].astype(bf), p['Wt_o'][1].astype(bf),
      p['ln3_s'][1].reshape(1, _D), p['ln3_b'][1].reshape(1, _D),
      p['W_mlp1'][1].astype(bf), p['b_mlp1'][1].reshape(1, 4 * _D),
      p['W_mlp2'][1].astype(bf), p['b_mlp2'][1].reshape(1, _D),
      p['W_out'].astype(bf), p['b_out'].reshape(1, _LATENT),
      p['codebook'], wact_pad.astype(bf))

    return out28[:, :_ADIM].reshape(_B, _T - 1, 1, _ADIM)


# speedup vs baseline: 1.4030x; 1.0531x over previous
import functools

import jax
import jax.numpy as jnp
from jax.experimental import pallas as pl
from jax.experimental.pallas import tpu as pltpu

_IN_DIM = 3
_D = 512
_LATENT = 32
_NLAT = 8192
_PATCH = 16
_NBLK = 2
_NH = 8
_ADIM = 7
_B, _T, _H, _W = 4, 8, 224, 224
_PTD = _IN_DIM * _PATCH * _PATCH
_N = (_H // _PATCH) * (_W // _PATCH) + 1
_HD = _D // _NH
_SCALE = _HD ** -0.5


def _dot(a, b):
    return jax.lax.dot_general(
        a.astype(jnp.bfloat16), b.astype(jnp.bfloat16),
        (((a.ndim - 1,), (0,)), ((), ())),
        preferred_element_type=jnp.float32)


def _dot_nt(a, b):
    return jax.lax.dot_general(
        a.astype(jnp.bfloat16), b.astype(jnp.bfloat16),
        (((1,), (1,)), ((), ())),
        preferred_element_type=jnp.float32)


def _ln(x, s, b):
    mu = x.mean(-1, keepdims=True)
    var = ((x - mu) ** 2).mean(-1, keepdims=True)
    return (x - mu) / jnp.sqrt(var + 1e-6) * s + b


def _bf(x):
    return x.astype(jnp.bfloat16).astype(jnp.float32)


def _in_proj_body(p_ref, w_ref, b_ref, o_ref):
    p = p_ref[0, 0]
    o_ref[0, 0] = _dot(p, w_ref[...]) + b_ref[0]


def _spatial_body(x_ref, s_ref, bb_ref, wqkv_ref, wo_ref, o_ref):
    x = x_ref[0, 0]
    h = _ln(x, s_ref[0], bb_ref[0])
    qkv = _dot(h, wqkv_ref[...])
    outs = []
    for hh in range(_NH):
        q = qkv[:, hh * _HD:(hh + 1) * _HD]
        k = qkv[:, _D + hh * _HD:_D + (hh + 1) * _HD]
        v = qkv[:, 2 * _D + hh * _HD:2 * _D + (hh + 1) * _HD]
        s = _dot_nt(q, k) * _SCALE
        a = jax.nn.softmax(s, axis=-1)
        outs.append(_dot(a, v))
    o = jnp.concatenate(outs, axis=1)
    o_ref[0, 0] = x + _dot(o, wo_ref[...])


def _fold(f, to):
    while f.shape[-1] > to:
        w = f.shape[-1] // 2
        f = f[:, :w] + f[:, w:]
    return f


def _foldmax(f, to):
    while f.shape[-1] > to:
        w = f.shape[-1] // 2
        f = jnp.maximum(f[:, :w], f[:, w:])
    return f


def _tile(f, to):
    while f.shape[-1] < to:
        f = jnp.concatenate([f, f], axis=1)
    return f


def _temporal_body(x_ref, s_ref, bb_ref, wqkv_ref, wo_ref, o_ref):
    x = x_ref[0]
    h = _ln(x, s_ref[0], bb_ref[0])
    qkv = _dot(h.reshape(_T * _N, _D), wqkv_ref[...]).reshape(_T, _N, 3 * _D)
    r = _bf(qkv)
    q, k, v = r[:, :, :_D], r[:, :, _D:2 * _D], r[:, :, 2 * _D:]
    neg = jnp.full((_N, _NH), -1e9, jnp.float32)
    o_rows = []
    for t in range(_T):
        cols = [_fold(q[t] * k[u], _NH) for u in range(t + 1)]
        s = jnp.concatenate(cols + [neg] * (_T - t - 1), axis=1) * _SCALE
        m = _tile(_foldmax(s, _NH), _T * _NH)
        e = jnp.exp(s - m)
        a = e / _tile(_fold(e, _NH), _T * _NH)
        ab = _bf(a)
        o_t = jnp.zeros((_N, _D), jnp.float32)
        for u in range(t + 1):
            o_t = o_t + _tile(ab[:, 8 * u:8 * u + 8], _D) * v[u]
        o_rows.append(o_t)
    o = jnp.stack(o_rows).reshape(_T * _N, _D)
    o_ref[0] = x + _dot(o, wo_ref[...]).reshape(_T, _N, _D)


def _mlp_body(x_ref, s_ref, bb_ref, w1_ref, b1_ref, w2_ref, b2_ref, o_ref):
    x = x_ref[0].reshape(_T * _N, _D)
    h = _ln(x, s_ref[0], bb_ref[0])
    a = jax.nn.gelu(_dot(h, w1_ref[...]) + b1_ref[0])
    o_ref[0] = (x + _dot(a, w2_ref[...]) + b2_ref[0]).reshape(_T, _N, _D)


def _spatial_q0_body(x_ref, s_ref, bb_ref, wq_ref, wkv_ref, wo_ref, o_ref):
    x = x_ref[0, 0]
    h = _ln(x, s_ref[0], bb_ref[0])
    kv = _dot(h, wkv_ref[...])
    q0 = _dot(h[0:1], wq_ref[...])
    outs = []
    for hh in range(_NH):
        q = q0[:, hh * _HD:(hh + 1) * _HD]
        k = kv[:, hh * _HD:(hh + 1) * _HD]
        v = kv[:, _D + hh * _HD:_D + (hh + 1) * _HD]
        s = _dot_nt(q, k) * _SCALE
        a = jax.nn.softmax(s, axis=-1)
        outs.append(_dot(a, v))
    o = jnp.concatenate(outs, axis=1)
    o_ref[0, 0] = x[0:1] + _dot(o, wo_ref[...])


def _tail_body(xa_ref, pen_ref, s2_ref, b2_ref, wqkv_ref, wo_ref,
               s3_ref, b3_ref, w1_ref, bm1_ref, w2_ref, bm2_ref,
               wout_ref, bout_ref, cb_ref, wact_ref, out_ref):
    x = xa_ref[...].reshape(_B * _T, _D)
    h = _ln(x, s2_ref[0], b2_ref[0])
    qkv = _dot(h, wqkv_ref[...])
    pen = pen_ref[...]
    o_rows = []
    for b in range(_B):
        qkv_b = qkv[b * _T:(b + 1) * _T]
        outs = []
        for hh in range(_NH):
            q = qkv_b[:, hh * _HD:(hh + 1) * _HD]
            k = qkv_b[:, _D + hh * _HD:_D + (hh + 1) * _HD]
            v = qkv_b[:, 2 * _D + hh * _HD:2 * _D + (hh + 1) * _HD]
            s = _dot_nt(q, k) * _SCALE + pen
            a = jax.nn.softmax(s, axis=-1)
            outs.append(_dot(a, v))
        o_rows.append(jnp.concatenate(outs, axis=1))
    o = jnp.concatenate(o_rows, axis=0)
    x2 = x + _dot(o, wo_ref[...])
    h3 = _ln(x2, s3_ref[0], b3_ref[0])
    m = jax.nn.gelu(_dot(h3, w1_ref[...]) + bm1_ref[0])
    x3 = x2 + _dot(m, w2_ref[...]) + bm2_ref[0]
    z = _dot(x3, wout_ref[...]) + bout_ref[0]
    z28 = jnp.concatenate(
        [z[b * _T + 1:(b + 1) * _T] for b in range(_B)], axis=0)
    cb = cb_ref[...]
    flat_sq = jnp.sum(z28 * z28, axis=-1, keepdims=True)
    cb_sq = jnp.transpose(jnp.sum(cb * cb, axis=-1, keepdims=True))
    d = (flat_sq - 2.0 * _dot_nt(z28, cb)) + cb_sq
    ii = jax.lax.broadcasted_iota(jnp.int32, (28, _NLAT), 1)
    dmin = jnp.min(d, axis=-1, keepdims=True)
    idx = jnp.min(jnp.where(d == dmin, ii, _NLAT), axis=-1)
    oh = (ii == idx[:, None]).astype(jnp.float32)
    zq = _dot(oh, cb)
    out_ref[...] = _dot(zq, wact_ref[...])


def _patches(videos):
    b, t, h, w, c = videos.shape
    hp, wp = h // _PATCH, w // _PATCH
    x = videos.reshape(b, t, hp, _PATCH, wp, _PATCH, c)
    x = x.transpose(0, 1, 2, 4, 3, 5, 6)
    return x.reshape(b, t, hp * wp, _PATCH * _PATCH * c)


def _full_spec(shape):
    nd = len(shape)
    return pl.BlockSpec(shape, lambda *_: (0,) * nd)


def kernel(videos, params):
    p = params
    bf = jnp.bfloat16
    patches = _patches(videos)
    act = jnp.broadcast_to(p['action_in'], (_B, _T, 1, _PTD))
    pin = jnp.concatenate([act, patches], axis=2)

    ar = jnp.arange(_D)
    perm = (ar % _NH) * _HD + ar // _NH
    qkv_perm = jnp.concatenate([perm, _D + perm, 2 * _D + perm])

    row_spec = pl.BlockSpec((1, 1, _N, _PTD), lambda b, t: (b, t, 0, 0))
    x_spec = pl.BlockSpec((1, 1, _N, _D), lambda b, t: (b, t, 0, 0))
    xt_spec = pl.BlockSpec((1, _T, _N, _D), lambda b: (b, 0, 0, 0))

    x = pl.pallas_call(
        _in_proj_body,
        grid=(_B, _T),
        in_specs=[row_spec,
                  _full_spec((_PTD, _D)),
                  _full_spec((1, _D))],
        out_specs=x_spec,
        out_shape=jax.ShapeDtypeStruct((_B, _T, _N, _D), jnp.float32),
    )(pin, p['W_in'].astype(bf), p['b_in'].reshape(1, _D))

    def spatial(x, i):
        return pl.pallas_call(
            _spatial_body,
            grid=(_B, _T),
            in_specs=[x_spec,
                      _full_spec((1, _D)), _full_spec((1, _D)),
                      _full_spec((_D, 3 * _D)), _full_spec((_D, _D))],
            out_specs=x_spec,
            out_shape=jax.ShapeDtypeStruct((_B, _T, _N, _D), jnp.float32),
        )(x, p['ln1_s'][i].reshape(1, _D), p['ln1_b'][i].reshape(1, _D),
          p['Ws_qkv'][i].astype(bf), p['Ws_o'][i].astype(bf))

    def temporal(x, i):
        return pl.pallas_call(
            _temporal_body,
            grid=(_B,),
            in_specs=[xt_spec,
                      _full_spec((1, _D)), _full_spec((1, _D)),
                      _full_spec((_D, 3 * _D)), _full_spec((_D, _D))],
            out_specs=xt_spec,
            out_shape=jax.ShapeDtypeStruct((_B, _T, _N, _D), jnp.float32),
        )(x, p['ln2_s'][i].reshape(1, _D), p['ln2_b'][i].reshape(1, _D),
          p['Wt_qkv'][i][:, qkv_perm].astype(bf),
          p['Wt_o'][i][perm, :].astype(bf))

    def mlp(x, i):
        return pl.pallas_call(
            _mlp_body,
            grid=(_B,),
            in_specs=[xt_spec,
                      _full_spec((1, _D)), _full_spec((1, _D)),
                      _full_spec((_D, 4 * _D)), _full_spec((1, 4 * _D)),
                      _full_spec((4 * _D, _D)), _full_spec((1, _D))],
            out_specs=xt_spec,
            out_shape=jax.ShapeDtypeStruct((_B, _T, _N, _D), jnp.float32),
        )(x, p['ln3_s'][i].reshape(1, _D), p['ln3_b'][i].reshape(1, _D),
          p['W_mlp1'][i].astype(bf), p['b_mlp1'][i].reshape(1, 4 * _D),
          p['W_mlp2'][i].astype(bf), p['b_mlp2'][i].reshape(1, _D))

    x = spatial(x, 0)
    x = temporal(x, 0)
    x = mlp(x, 0)

    xa = pl.pallas_call(
        _spatial_q0_body,
        grid=(_B, _T),
        in_specs=[x_spec,
                  _full_spec((1, _D)), _full_spec((1, _D)),
                  _full_spec((_D, _D)), _full_spec((_D, 2 * _D)),
                  _full_spec((_D, _D))],
        out_specs=pl.BlockSpec((1, 1, 1, _D), lambda b, t: (b, t, 0, 0)),
        out_shape=jax.ShapeDtypeStruct((_B, _T, 1, _D), jnp.float32),
    )(x, p['ln1_s'][1].reshape(1, _D), p['ln1_b'][1].reshape(1, _D),
      p['Ws_qkv'][1][:, :_D].astype(bf), p['Ws_qkv'][1][:, _D:].astype(bf),
      p['Ws_o'][1].astype(bf))

    wact_pad = jnp.zeros((_LATENT, 16), jnp.float32).at[:, :_ADIM].set(
        p['W_action'])
    pen = jnp.where(jnp.tril(jnp.ones((_T, _T), jnp.bool_)),
                    0.0, -1e9).astype(jnp.float32)
    out28 = pl.pallas_call(
        _tail_body,
        grid=(1,),
        in_specs=[_full_spec((_B, _T, 1, _D)),
                  _full_spec((_T, _T)),
                  _full_spec((1, _D)), _full_spec((1, _D)),
                  _full_spec((_D, 3 * _D)), _full_spec((_D, _D)),
                  _full_spec((1, _D)), _full_spec((1, _D)),
                  _full_spec((_D, 4 * _D)), _full_spec((1, 4 * _D)),
                  _full_spec((4 * _D, _D)), _full_spec((1, _D)),
                  _full_spec((_D, _LATENT)), _full_spec((1, _LATENT)),
                  _full_spec((_NLAT, _LATENT)), _full_spec((_LATENT, 16))],
        out_specs=_full_spec((28, 16)),
        out_shape=jax.ShapeDtypeStruct((28, 16), jnp.float32),
    )(xa, pen,
      p['ln2_s'][1].reshape(1, _D), p['ln2_b'][1].reshape(1, _D),
      p['Wt_qkv'][1].astype(bf), p['Wt_o'][1].astype(bf),
      p['ln3_s'][1].reshape(1, _D), p['ln3_b'][1].reshape(1, _D),
      p['W_mlp1'][1].astype(bf), p['b_mlp1'][1].reshape(1, 4 * _D),
      p['W_mlp2'][1].astype(bf), p['b_mlp2'][1].reshape(1, _D),
      p['W_out'].astype(bf), p['b_out'].reshape(1, _LATENT),
      p['codebook'], wact_pad.astype(bf))

    return out28[:, :_ADIM].reshape(_B, _T - 1, 1, _ADIM)
